# baseline (speedup 1.0000x reference)
import jax
import jax.numpy as jnp
from jax import lax
from jax.experimental import pallas as pl
from jax.experimental.pallas import tpu as pltpu
from jax.experimental.pallas import tpu_sc as plsc
from jax._src.pallas import mpmd as _plmpmd

N_CLASSES = 100000
D = 64
A_EMA = 0.5
B = 16384

_NC = 2
_NS = 16
_NW = _NC * _NS
_BPW = B // _NW
_CH = 128
_NCH = _BPW // _CH
_W = 16
_NCC = D // (_NC * _W)
_RPT = B // _NS
_NJ = _RPT // _CH

_SCP = pltpu.CompilerParams(use_tc_tiling_on_sc=False,
                            needs_layout_passes=False)


def _mesh():
    return plsc.VectorSubcoreMesh(core_axis_name="c", subcore_axis_name="s")


def _rsqrt16(ss):
    i = lax.bitcast_convert_type(ss, jnp.int32)
    y = lax.bitcast_convert_type(0x5F3759DF - (i >> 1), jnp.float32)
    for _ in range(3):
        y = y * (1.5 - 0.5 * ss * y * y)
    return y


def _front_body(feat_hbm, lab_hbm, tab_hbm, fn_hbm, bc_hbm, loss_hbm,
                idx_v, work_v, rows_v, acc_v, sem):
    wid = lax.axis_index("s") * _NC + lax.axis_index("c")
    base = wid * _BPW
    for j in range(_NCH):
        pltpu.sync_copy(lab_hbm.at[pl.ds(base + j * _CH, _CH)], idx_v.at[j])
    copies = []
    for j in range(_NCH):
        copies.append(
            pltpu.async_copy(tab_hbm.at[idx_v.at[j]],
                             rows_v.at[pl.ds(j * _CH, _CH)], sem))
    pltpu.sync_copy(feat_hbm.at[pl.ds(base, _BPW)], work_v)

    def _nrm(r, carry):
        v0 = work_v[r, pl.ds(0, 16)]
        v1 = work_v[r, pl.ds(16, 16)]
        v2 = work_v[r, pl.ds(32, 16)]
        v3 = work_v[r, pl.ds(48, 16)]
        ss = jnp.sum(v0 * v0 + v1 * v1 + v2 * v2 + v3 * v3)
        y = _rsqrt16(jnp.full((16,), ss, jnp.float32))
        work_v[r, pl.ds(0, 16)] = v0 * y
        work_v[r, pl.ds(16, 16)] = v1 * y
        work_v[r, pl.ds(32, 16)] = v2 * y
        work_v[r, pl.ds(48, 16)] = v3 * y
        return carry

    lax.fori_loop(0, _BPW, _nrm, 0)
    pltpu.sync_copy(work_v, fn_hbm.at[pl.ds(base, _BPW)])
    for cp in copies:
        cp.wait()
    pltpu.sync_copy(rows_v, bc_hbm.at[pl.ds(base, _BPW)])

    acc_v[0] = jnp.zeros((16,), jnp.float32)

    def _loss(r, carry):
        b0 = rows_v[r, pl.ds(0, 16)]
        b1 = rows_v[r, pl.ds(16, 16)]
        b2 = rows_v[r, pl.ds(32, 16)]
        b3 = rows_v[r, pl.ds(48, 16)]
        ss = jnp.sum(b0 * b0 + b1 * b1 + b2 * b2 + b3 * b3)
        y = _rsqrt16(jnp.full((16,), ss, jnp.float32))
        d0 = work_v[r, pl.ds(0, 16)] - b0 * y
        d1 = work_v[r, pl.ds(16, 16)] - b1 * y
        d2 = work_v[r, pl.ds(32, 16)] - b2 * y
        d3 = work_v[r, pl.ds(48, 16)] - b3 * y
        acc_v[0] = acc_v[0] + (d0 * d0 + d1 * d1 + d2 * d2 + d3 * d3)
        return carry

    lax.fori_loop(0, _BPW, _loss, 0)
    pltpu.sync_copy(acc_v, loss_hbm.at[pl.ds(wid, 1)])


def _front(feat, lab, tab):
    k = pl.kernel(
        _front_body,
        out_type=(jax.ShapeDtypeStruct((B, D), jnp.float32),
                  jax.ShapeDtypeStruct((B, D), jnp.float32),
                  jax.ShapeDtypeStruct((_NW, 16), jnp.float32)),
        mesh=_mesh(),
        compiler_params=_SCP,
        scratch_types=[
            pltpu.VMEM((_NCH, _CH), jnp.int32),
            pltpu.VMEM((_BPW, D), jnp.float32),
            pltpu.VMEM((_BPW, D), jnp.float32),
            pltpu.VMEM((1, 16), jnp.float32),
            pltpu.SemaphoreType.DMA,
        ],
    )
    return k(feat, lab, tab)


def _segsum_body(fn_hbm, lab_hbm, msum_hbm, cnt_hbm,
                 idx_v, src_v, zrow_v, zcnt_v, one_v, cbuf_v, spm, spmc):
    s = lax.axis_index("s")
    c = lax.axis_index("c")
    base = s * _RPT

    for j in range(_NJ):
        pltpu.sync_copy(lab_hbm.at[pl.ds(base + j * _CH, _CH)], idx_v.at[j])

    def _zrow(r, carry):
        zrow_v[r] = jnp.zeros((_W,), jnp.float32)
        return carry

    lax.fori_loop(0, _CH, _zrow, 0)
    for k in range(_CH // 16):
        zcnt_v[pl.ds(k * 16, 16)] = jnp.zeros((16,), jnp.float32)
        one_v[pl.ds(k * 16, 16)] = jnp.ones((16,), jnp.float32)

    for cc in range(_NCC):
        col = (c * _NCC + cc) * _W
        for j in range(_NJ):
            pltpu.sync_copy(zrow_v, spm.at[idx_v.at[j]])
        if cc == 0:
            @pl.when(c == 0)
            def _():
                for j in range(_NJ):
                    pltpu.sync_copy(zcnt_v, spmc.at[idx_v.at[j]])
        plsc.subcore_barrier()
        pltpu.sync_copy(fn_hbm.at[pl.ds(base, _RPT), pl.ds(col, _W)], src_v)
        for j in range(_NJ):
            pltpu.sync_copy(src_v.at[pl.ds(j * _CH, _CH)],
                            spm.at[idx_v.at[j]], add=True)
        if cc == 0:
            @pl.when(c == 0)
            def _():
                for j in range(_NJ):
                    pltpu.sync_copy(one_v, spmc.at[idx_v.at[j]], add=True)
        plsc.subcore_barrier()
        for j in range(_NJ):
            pltpu.sync_copy(spm.at[idx_v.at[j]],
                            src_v.at[pl.ds(j * _CH, _CH)])
        pltpu.sync_copy(src_v, msum_hbm.at[pl.ds(base, _RPT), pl.ds(col, _W)])
        if cc == 0:
            @pl.when(c == 0)
            def _():
                for j in range(_NJ):
                    pltpu.sync_copy(spmc.at[idx_v.at[j]],
                                    cbuf_v.at[pl.ds(j * _CH, _CH)])
                pltpu.sync_copy(cbuf_v, cnt_hbm.at[pl.ds(base, _RPT)])
        plsc.subcore_barrier()


def _segment_sums(fn, lab):
    k = pl.kernel(
        _segsum_body,
        out_type=(jax.ShapeDtypeStruct((B, D), jnp.float32),
                  jax.ShapeDtypeStruct((B,), jnp.float32)),
        mesh=_mesh(),
        compiler_params=_SCP,
        scratch_types=[
            pltpu.VMEM((_NJ, _CH), jnp.int32),
            pltpu.VMEM((_RPT, _W), jnp.float32),
            pltpu.VMEM((_CH, _W), jnp.float32),
            pltpu.VMEM((_CH,), jnp.float32),
            pltpu.VMEM((_CH,), jnp.float32),
            pltpu.VMEM((_RPT,), jnp.float32),
            pltpu.VMEM_SHARED((N_CLASSES, _W), jnp.float32),
            pltpu.VMEM_SHARED((N_CLASSES,), jnp.float32),
        ],
    )
    return k(fn, lab)


def _scatter_body(cen_hbm, idx_hbm, bc_hbm, ms_hbm, ct_hbm,
                  out_hbm, idx_v, bbuf, mbuf, cbuf, sem):
    del cen_hbm
    wid = lax.axis_index("s") * _NC + lax.axis_index("c")
    base = wid * _BPW
    for j in range(_NCH):
        pltpu.sync_copy(idx_hbm.at[pl.ds(base + j * _CH, _CH)], idx_v.at[j])
    pltpu.sync_copy(bc_hbm.at[pl.ds(base, _BPW)], bbuf)
    pltpu.sync_copy(ms_hbm.at[pl.ds(base, _BPW)], mbuf)
    pltpu.sync_copy(ct_hbm.at[pl.ds(base, _BPW)], cbuf)

    def _blend(r, carry):
        cv = plsc.load_gather(cbuf, [jnp.full((16,), r, jnp.int32)])
        rec = A_EMA / cv
        for k in range(D // 16):
            sl = pl.ds(k * 16, 16)
            mbuf[r, sl] = (1.0 - A_EMA) * bbuf[r, sl] + mbuf[r, sl] * rec
        return carry

    lax.fori_loop(0, _BPW, _blend, 0)

    copies = []
    for j in range(_NCH):
        copies.append(
            pltpu.async_copy(mbuf.at[pl.ds(j * _CH, _CH)],
                             out_hbm.at[idx_v.at[j]], sem))
    for cp in copies:
        cp.wait()


def _scatter(cen, idx, bc, msum, cnt):
    k = _plmpmd._mpmd_map(
        [(_mesh(), _scatter_body)],
        jax.ShapeDtypeStruct((N_CLASSES, D), jnp.float32),
        input_output_aliases={0: 0},
        compiler_params=_SCP,
        scratch_types=[
            pltpu.VMEM((_NCH, _CH), jnp.int32),
            pltpu.VMEM((_BPW, D), jnp.float32),
            pltpu.VMEM((_BPW, D), jnp.float32),
            pltpu.VMEM((_BPW,), jnp.float32),
            pltpu.SemaphoreType.DMA,
        ],
    )
    return k(cen, idx, bc, msum, cnt)


def kernel(features, labels, centers):
    labels = labels.astype(jnp.int32)
    fn, bc, loss_p = _front(features, labels, centers)
    msum, cnt = _segment_sums(fn, labels)
    new_centers = _scatter(centers, labels, bc, msum, cnt)
    loss = jnp.sum(loss_p) / (2.0 * B)
    return loss, new_centers


# speedup vs baseline: 1.0709x; 1.0709x over previous
import jax
import jax.numpy as jnp
from jax import lax
from jax.experimental import pallas as pl
from jax.experimental.pallas import tpu as pltpu
from jax.experimental.pallas import tpu_sc as plsc
from jax._src.pallas import mpmd as _plmpmd

N_CLASSES = 100000
D = 64
A_EMA = 0.5
B = 16384

_NC = 2
_NS = 16
_NW = _NC * _NS
_BPW = B // _NW
_CH = 128
_NCH = _BPW // _CH
_W = 16
_NCC = D // (_NC * _W)
_RPT = B // _NS
_NJ = _RPT // _CH

_SCP = pltpu.CompilerParams(use_tc_tiling_on_sc=False,
                            needs_layout_passes=False)


def _mesh():
    return plsc.VectorSubcoreMesh(core_axis_name="c", subcore_axis_name="s")


def _rsqrt16(ss):
    i = lax.bitcast_convert_type(ss, jnp.int32)
    y = lax.bitcast_convert_type(0x5F3759DF - (i >> 1), jnp.float32)
    for _ in range(3):
        y = y * (1.5 - 0.5 * ss * y * y)
    return y


def _norm_body(feat_hbm, fn_hbm, work_v):
    wid = lax.axis_index("s") * _NC + lax.axis_index("c")
    base = wid * _BPW
    pltpu.sync_copy(feat_hbm.at[pl.ds(base, _BPW)], work_v)

    def _nrm(r, carry):
        v0 = work_v[r, pl.ds(0, 16)]
        v1 = work_v[r, pl.ds(16, 16)]
        v2 = work_v[r, pl.ds(32, 16)]
        v3 = work_v[r, pl.ds(48, 16)]
        ss = jnp.sum(v0 * v0 + v1 * v1 + v2 * v2 + v3 * v3)
        y = _rsqrt16(jnp.full((16,), ss, jnp.float32))
        work_v[r, pl.ds(0, 16)] = v0 * y
        work_v[r, pl.ds(16, 16)] = v1 * y
        work_v[r, pl.ds(32, 16)] = v2 * y
        work_v[r, pl.ds(48, 16)] = v3 * y
        return carry

    lax.fori_loop(0, _BPW, _nrm, 0)
    pltpu.sync_copy(work_v, fn_hbm.at[pl.ds(base, _BPW)])


def _normalize(feat):
    k = pl.kernel(
        _norm_body,
        out_type=jax.ShapeDtypeStruct((B, D), jnp.float32),
        mesh=_mesh(),
        compiler_params=_SCP,
        scratch_types=[pltpu.VMEM((_BPW, D), jnp.float32)],
    )
    return k(feat)


def _gl_body(tab_hbm, lab_hbm, fn_hbm, bc_hbm, loss_hbm,
             idx_v, work_v, rows_v, acc_v, sem):
    wid = lax.axis_index("s") * _NC + lax.axis_index("c")
    base = wid * _BPW
    for j in range(_NCH):
        pltpu.sync_copy(lab_hbm.at[pl.ds(base + j * _CH, _CH)], idx_v.at[j])
    copies = []
    for j in range(_NCH):
        copies.append(
            pltpu.async_copy(tab_hbm.at[idx_v.at[j]],
                             rows_v.at[pl.ds(j * _CH, _CH)], sem))
    pltpu.sync_copy(fn_hbm.at[pl.ds(base, _BPW)], work_v)
    for cp in copies:
        cp.wait()
    pltpu.sync_copy(rows_v, bc_hbm.at[pl.ds(base, _BPW)])

    acc_v[0] = jnp.zeros((16,), jnp.float32)

    def _loss(r, carry):
        b0 = rows_v[r, pl.ds(0, 16)]
        b1 = rows_v[r, pl.ds(16, 16)]
        b2 = rows_v[r, pl.ds(32, 16)]
        b3 = rows_v[r, pl.ds(48, 16)]
        ss = jnp.sum(b0 * b0 + b1 * b1 + b2 * b2 + b3 * b3)
        y = _rsqrt16(jnp.full((16,), ss, jnp.float32))
        d0 = work_v[r, pl.ds(0, 16)] - b0 * y
        d1 = work_v[r, pl.ds(16, 16)] - b1 * y
        d2 = work_v[r, pl.ds(32, 16)] - b2 * y
        d3 = work_v[r, pl.ds(48, 16)] - b3 * y
        acc_v[0] = acc_v[0] + (d0 * d0 + d1 * d1 + d2 * d2 + d3 * d3)
        return carry

    lax.fori_loop(0, _BPW, _loss, 0)
    pltpu.sync_copy(acc_v, loss_hbm.at[pl.ds(wid, 1)])


def _gather_loss(tab, lab, fn):
    k = pl.kernel(
        _gl_body,
        out_type=(jax.ShapeDtypeStruct((B, D), jnp.float32),
                  jax.ShapeDtypeStruct((_NW, 16), jnp.float32)),
        mesh=_mesh(),
        compiler_params=_SCP,
        scratch_types=[
            pltpu.VMEM((_NCH, _CH), jnp.int32),
            pltpu.VMEM((_BPW, D), jnp.float32),
            pltpu.VMEM((_BPW, D), jnp.float32),
            pltpu.VMEM((1, 16), jnp.float32),
            pltpu.SemaphoreType.DMA,
        ],
    )
    return k(tab, lab, fn)


def _segsum_body(fn_hbm, lab_hbm, msum_hbm, cnt_hbm,
                 idx_v, src_v, zrow_v, zcnt_v, one_v, cbuf_v, spm, spmc):
    s = lax.axis_index("s")
    c = lax.axis_index("c")
    base = s * _RPT

    for j in range(_NJ):
        pltpu.sync_copy(lab_hbm.at[pl.ds(base + j * _CH, _CH)], idx_v.at[j])

    def _zrow(r, carry):
        zrow_v[r] = jnp.zeros((_W,), jnp.float32)
        return carry

    lax.fori_loop(0, _CH, _zrow, 0)
    for k in range(_CH // 16):
        zcnt_v[pl.ds(k * 16, 16)] = jnp.zeros((16,), jnp.float32)
        one_v[pl.ds(k * 16, 16)] = jnp.ones((16,), jnp.float32)

    for cc in range(_NCC):
        col = (c * _NCC + cc) * _W
        for j in range(_NJ):
            pltpu.sync_copy(zrow_v, spm.at[idx_v.at[j]])
        if cc == 0:
            @pl.when(c == 0)
            def _():
                for j in range(_NJ):
                    pltpu.sync_copy(zcnt_v, spmc.at[idx_v.at[j]])
        plsc.subcore_barrier()
        pltpu.sync_copy(fn_hbm.at[pl.ds(base, _RPT), pl.ds(col, _W)], src_v)
        for j in range(_NJ):
            pltpu.sync_copy(src_v.at[pl.ds(j * _CH, _CH)],
                            spm.at[idx_v.at[j]], add=True)
        if cc == 0:
            @pl.when(c == 0)
            def _():
                for j in range(_NJ):
                    pltpu.sync_copy(one_v, spmc.at[idx_v.at[j]], add=True)
        plsc.subcore_barrier()
        for j in range(_NJ):
            pltpu.sync_copy(spm.at[idx_v.at[j]],
                            src_v.at[pl.ds(j * _CH, _CH)])
        pltpu.sync_copy(src_v, msum_hbm.at[pl.ds(base, _RPT), pl.ds(col, _W)])
        if cc == 0:
            @pl.when(c == 0)
            def _():
                for j in range(_NJ):
                    pltpu.sync_copy(spmc.at[idx_v.at[j]],
                                    cbuf_v.at[pl.ds(j * _CH, _CH)])
                pltpu.sync_copy(cbuf_v, cnt_hbm.at[pl.ds(base, _RPT)])
        plsc.subcore_barrier()


def _segment_sums(fn, lab):
    k = pl.kernel(
        _segsum_body,
        out_type=(jax.ShapeDtypeStruct((B, D), jnp.float32),
                  jax.ShapeDtypeStruct((B,), jnp.float32)),
        mesh=_mesh(),
        compiler_params=_SCP,
        scratch_types=[
            pltpu.VMEM((_NJ, _CH), jnp.int32),
            pltpu.VMEM((_RPT, _W), jnp.float32),
            pltpu.VMEM((_CH, _W), jnp.float32),
            pltpu.VMEM((_CH,), jnp.float32),
            pltpu.VMEM((_CH,), jnp.float32),
            pltpu.VMEM((_RPT,), jnp.float32),
            pltpu.VMEM_SHARED((N_CLASSES, _W), jnp.float32),
            pltpu.VMEM_SHARED((N_CLASSES,), jnp.float32),
        ],
    )
    return k(fn, lab)


def _scatter_body(cen_hbm, idx_hbm, bc_hbm, ms_hbm, ct_hbm,
                  out_hbm, idx_v, bbuf, mbuf, cbuf, sem):
    del cen_hbm
    wid = lax.axis_index("s") * _NC + lax.axis_index("c")
    base = wid * _BPW
    for j in range(_NCH):
        pltpu.sync_copy(idx_hbm.at[pl.ds(base + j * _CH, _CH)], idx_v.at[j])
    pltpu.sync_copy(bc_hbm.at[pl.ds(base, _BPW)], bbuf)
    pltpu.sync_copy(ms_hbm.at[pl.ds(base, _BPW)], mbuf)
    pltpu.sync_copy(ct_hbm.at[pl.ds(base, _BPW)], cbuf)

    def _blend(r, carry):
        cv = plsc.load_gather(cbuf, [jnp.full((16,), r, jnp.int32)])
        rec = A_EMA / cv
        for k in range(D // 16):
            sl = pl.ds(k * 16, 16)
            mbuf[r, sl] = (1.0 - A_EMA) * bbuf[r, sl] + mbuf[r, sl] * rec
        return carry

    lax.fori_loop(0, _BPW, _blend, 0)

    copies = []
    for j in range(_NCH):
        copies.append(
            pltpu.async_copy(mbuf.at[pl.ds(j * _CH, _CH)],
                             out_hbm.at[idx_v.at[j]], sem))
    for cp in copies:
        cp.wait()


def _scatter(cen, idx, bc, msum, cnt):
    k = _plmpmd._mpmd_map(
        [(_mesh(), _scatter_body)],
        jax.ShapeDtypeStruct((N_CLASSES, D), jnp.float32),
        input_output_aliases={0: 0},
        compiler_params=_SCP,
        scratch_types=[
            pltpu.VMEM((_NCH, _CH), jnp.int32),
            pltpu.VMEM((_BPW, D), jnp.float32),
            pltpu.VMEM((_BPW, D), jnp.float32),
            pltpu.VMEM((_BPW,), jnp.float32),
            pltpu.SemaphoreType.DMA,
        ],
    )
    return k(cen, idx, bc, msum, cnt)


def kernel(features, labels, centers):
    labels = labels.astype(jnp.int32)
    fn = _normalize(features)
    msum, cnt = _segment_sums(fn, labels)
    bc, loss_p = _gather_loss(centers, labels, fn)
    new_centers = _scatter(centers, labels, bc, msum, cnt)
    loss = jnp.sum(loss_p) / (2.0 * B)
    return loss, new_centers


# speedup vs baseline: 1.1696x; 1.0921x over previous
import functools

import jax
import jax.numpy as jnp
from jax import lax
from jax.experimental import pallas as pl
from jax.experimental.pallas import tpu as pltpu
from jax.experimental.pallas import tpu_sc as plsc
from jax._src.pallas import mpmd as _plmpmd

N_CLASSES = 100000
D = 64
A_EMA = 0.5
B = 16384
EPS = 1e-12


def _norm_body(x_ref, o_ref):
    x = x_ref[...]
    n = jnp.sqrt(jnp.sum(x * x, axis=1, keepdims=True))
    o_ref[...] = x / jnp.maximum(n, EPS)


def _normalize(x):
    return pl.pallas_call(
        _norm_body,
        out_shape=jax.ShapeDtypeStruct((B, D), jnp.float32),
    )(x)


_NC = 2
_NS = 16
_NW = _NC * _NS
_BPW = B // _NW
_CH = 128
_NCH = _BPW // _CH


def _gather_body(tab_hbm, idx_hbm, out_hbm, idx_v, rows_v, sem):
    wid = lax.axis_index("s") * _NC + lax.axis_index("c")
    base = wid * _BPW
    for j in range(_NCH):
        pltpu.sync_copy(idx_hbm.at[pl.ds(base + j * _CH, _CH)], idx_v.at[j])
    copies = []
    for j in range(_NCH):
        copies.append(
            pltpu.async_copy(tab_hbm.at[idx_v.at[j]],
                             rows_v.at[pl.ds(j * _CH, _CH)], sem))
    for c in copies:
        c.wait()
    pltpu.sync_copy(rows_v, out_hbm.at[pl.ds(base, _BPW)])


def _gather_rows(tab, idx):
    k = pl.kernel(
        _gather_body,
        out_type=jax.ShapeDtypeStruct((B, D), jnp.float32),
        mesh=plsc.VectorSubcoreMesh(core_axis_name="c", subcore_axis_name="s"),
        compiler_params=pltpu.CompilerParams(use_tc_tiling_on_sc=False),
        scratch_types=[
            pltpu.VMEM((_NCH, _CH), jnp.int32),
            pltpu.VMEM((_BPW, D), jnp.float32),
            pltpu.SemaphoreType.DMA,
        ],
    )
    return k(tab, idx)


_W = 16
_NCC = D // (_NC * _W)
_RPT = B // _NS


def _segsum_body(fn_hbm, lab_hbm, msum_hbm, cnt_hbm,
                 idx_v, src_v, zrow_v, zcnt_v, one_v, cbuf_v, spm, spmc):
    s = lax.axis_index("s")
    c = lax.axis_index("c")
    base = s * _RPT
    nj = _RPT // _CH

    for j in range(nj):
        pltpu.sync_copy(lab_hbm.at[pl.ds(base + j * _CH, _CH)], idx_v.at[j])

    def _zrow(r, carry):
        zrow_v[r] = jnp.zeros((_W,), jnp.float32)
        return carry

    lax.fori_loop(0, _CH, _zrow, 0)
    for k in range(_CH // 16):
        zcnt_v[pl.ds(k * 16, 16)] = jnp.zeros((16,), jnp.float32)
        one_v[pl.ds(k * 16, 16)] = jnp.ones((16,), jnp.float32)

    for cc in range(_NCC):
        col = c * _NCC * _W + cc * _W
        for j in range(nj):
            pltpu.sync_copy(zrow_v, spm.at[idx_v.at[j]])
        if cc == 0:
            @pl.when(c == 0)
            def _():
                for j in range(nj):
                    pltpu.sync_copy(zcnt_v, spmc.at[idx_v.at[j]])
        plsc.subcore_barrier()
        pltpu.sync_copy(fn_hbm.at[pl.ds(base, _RPT), pl.ds(col, _W)], src_v)
        for j in range(nj):
            pltpu.sync_copy(src_v.at[pl.ds(j * _CH, _CH)],
                            spm.at[idx_v.at[j]], add=True)
        if cc == 0:
            @pl.when(c == 0)
            def _():
                for j in range(nj):
                    pltpu.sync_copy(one_v, spmc.at[idx_v.at[j]], add=True)
        plsc.subcore_barrier()
        for j in range(nj):
            pltpu.sync_copy(spm.at[idx_v.at[j]],
                            src_v.at[pl.ds(j * _CH, _CH)])
        pltpu.sync_copy(src_v, msum_hbm.at[pl.ds(base, _RPT), pl.ds(col, _W)])
        if cc == 0:
            @pl.when(c == 0)
            def _():
                for j in range(nj):
                    pltpu.sync_copy(spmc.at[idx_v.at[j]], cbuf_v.at[pl.ds(j * _CH, _CH)])
                pltpu.sync_copy(cbuf_v, cnt_hbm.at[pl.ds(base, _RPT)])
        plsc.subcore_barrier()


def _segment_sums(fn, lab):
    k = pl.kernel(
        _segsum_body,
        out_type=(jax.ShapeDtypeStruct((B, D), jnp.float32),
                  jax.ShapeDtypeStruct((B,), jnp.float32)),
        mesh=plsc.VectorSubcoreMesh(core_axis_name="c", subcore_axis_name="s"),
        compiler_params=pltpu.CompilerParams(use_tc_tiling_on_sc=False),
        scratch_types=[
            pltpu.VMEM((_RPT // _CH, _CH), jnp.int32),
            pltpu.VMEM((_RPT, _W), jnp.float32),
            pltpu.VMEM((_CH, _W), jnp.float32),
            pltpu.VMEM((_CH,), jnp.float32),
            pltpu.VMEM((_CH,), jnp.float32),
            pltpu.VMEM((_RPT,), jnp.float32),
            pltpu.VMEM_SHARED((N_CLASSES, _W), jnp.float32),
            pltpu.VMEM_SHARED((N_CLASSES,), jnp.float32),
        ],
    )
    return k(fn, lab)


def _epi_body(fn_ref, bc_ref, ms_ref, ct_ref, u_ref, loss_ref):
    craw = bc_ref[...]
    mean = ms_ref[...] / ct_ref[...]
    u_ref[...] = (1.0 - A_EMA) * craw + A_EMA * mean
    ss = jnp.sum(craw * craw, axis=1, keepdims=True)
    cn = craw / jnp.maximum(jnp.sqrt(ss), EPS)
    dlt = fn_ref[...] - cn
    loss_ref[...] = jnp.sum(dlt * dlt)[None, None]


def _epilogue(fn, bc, msum, cnt):
    return pl.pallas_call(
        _epi_body,
        out_shape=[
            jax.ShapeDtypeStruct((B, D), jnp.float32),
            jax.ShapeDtypeStruct((1, 1), jnp.float32),
        ],
    )(fn, bc, msum, cnt.reshape(B, 1))


def _scatter_body(cen_hbm, idx_hbm, u_hbm, out_hbm, idx_v, rows_v, sem):
    del cen_hbm
    wid = lax.axis_index("s") * _NC + lax.axis_index("c")
    base = wid * _BPW
    pltpu.sync_copy(idx_hbm.at[pl.ds(base, _BPW)], idx_v)
    pltpu.sync_copy(u_hbm.at[pl.ds(base, _BPW)], rows_v)

    def _row(r, carry):
        lv = plsc.load_gather(idx_v, [jnp.full((16,), r, jnp.int32)])
        lab = jnp.max(lv)
        pltpu.async_copy(rows_v.at[pl.ds(r, 1)],
                         out_hbm.at[pl.ds(lab, 1)], sem)
        return carry

    lax.fori_loop(0, _BPW, _row, 0)
    pltpu.make_async_copy(u_hbm.at[pl.ds(base, _BPW)], rows_v, sem).wait()


def _copy_scatter(cen, idx, u):
    k = _plmpmd._mpmd_map(
        [(plsc.VectorSubcoreMesh(core_axis_name="c", subcore_axis_name="s"),
          _scatter_body)],
        jax.ShapeDtypeStruct((N_CLASSES, D), jnp.float32),
        input_output_aliases={0: 0},
        compiler_params=pltpu.CompilerParams(needs_layout_passes=False),
        scratch_types=[
            pltpu.VMEM((_BPW,), jnp.int32),
            pltpu.VMEM((_BPW, D), jnp.float32),
            pltpu.SemaphoreType.DMA,
        ],
    )
    return k(cen, idx, u)


def kernel(features, labels, centers):
    labels = labels.astype(jnp.int32)
    fn = _normalize(features)
    bc = _gather_rows(centers, labels)
    msum, cnt = _segment_sums(fn, labels)
    u, loss2 = _epilogue(fn, bc, msum, cnt)
    new_centers = _copy_scatter(centers, labels, u)
    loss = loss2[0, 0] / (2.0 * B)
    return loss, new_centers


# speedup vs baseline: 1.4934x; 1.2769x over previous
import functools

import jax
import jax.numpy as jnp
from jax import lax
from jax.experimental import pallas as pl
from jax.experimental.pallas import tpu as pltpu
from jax.experimental.pallas import tpu_sc as plsc
from jax._src.pallas import mpmd as _plmpmd

N_CLASSES = 100000
D = 64
A_EMA = 0.5
B = 16384
EPS = 1e-12


def _norm_body(x_ref, o_ref):
    x = x_ref[...]
    n = jnp.sqrt(jnp.sum(x * x, axis=1, keepdims=True))
    o_ref[...] = x / jnp.maximum(n, EPS)


def _normalize(x):
    return pl.pallas_call(
        _norm_body,
        out_shape=jax.ShapeDtypeStruct((B, D), jnp.float32),
    )(x)


_NC = 2
_NS = 16
_NW = _NC * _NS
_BPW = B // _NW
_CH = 128
_NCH = _BPW // _CH


def _gather_body(tab_hbm, idx_hbm, out_hbm, idx_v, rows_v, sem):
    wid = lax.axis_index("s") * _NC + lax.axis_index("c")
    base = wid * _BPW
    pltpu.sync_copy(idx_hbm.at[pl.ds(base, _BPW)], idx_v)

    def _row(r, carry):
        lv = plsc.load_gather(idx_v, [jnp.full((16,), r, jnp.int32)])
        lab = jnp.max(lv)
        pltpu.async_copy(tab_hbm.at[pl.ds(lab, 1)],
                         rows_v.at[pl.ds(r, 1)], sem)
        return carry

    lax.fori_loop(0, _BPW, _row, 0)
    pltpu.make_async_copy(tab_hbm.at[pl.ds(0, _BPW)], rows_v, sem).wait()
    pltpu.sync_copy(rows_v, out_hbm.at[pl.ds(base, _BPW)])


def _gather_rows(tab, idx):
    k = pl.kernel(
        _gather_body,
        out_type=jax.ShapeDtypeStruct((B, D), jnp.float32),
        mesh=plsc.VectorSubcoreMesh(core_axis_name="c", subcore_axis_name="s"),
        compiler_params=pltpu.CompilerParams(needs_layout_passes=False),
        scratch_types=[
            pltpu.VMEM((_BPW,), jnp.int32),
            pltpu.VMEM((_BPW, D), jnp.float32),
            pltpu.SemaphoreType.DMA,
        ],
    )
    return k(tab, idx)


_W = 16
_NCC = D // (_NC * _W)
_RPT = B // _NS


def _segsum_body(fn_hbm, lab_hbm, msum_hbm, cnt_hbm,
                 idx_v, src_v, zrow_v, zcnt_v, one_v, cbuf_v, spm, spmc):
    s = lax.axis_index("s")
    c = lax.axis_index("c")
    base = s * _RPT
    nj = _RPT // _CH

    for j in range(nj):
        pltpu.sync_copy(lab_hbm.at[pl.ds(base + j * _CH, _CH)], idx_v.at[j])

    def _zrow(r, carry):
        zrow_v[r] = jnp.zeros((_W,), jnp.float32)
        return carry

    lax.fori_loop(0, _CH, _zrow, 0)
    for k in range(_CH // 16):
        zcnt_v[pl.ds(k * 16, 16)] = jnp.zeros((16,), jnp.float32)
        one_v[pl.ds(k * 16, 16)] = jnp.ones((16,), jnp.float32)

    for cc in range(_NCC):
        col = c * _NCC * _W + cc * _W
        for j in range(nj):
            pltpu.sync_copy(zrow_v, spm.at[idx_v.at[j]])
        if cc == 0:
            @pl.when(c == 0)
            def _():
                for j in range(nj):
                    pltpu.sync_copy(zcnt_v, spmc.at[idx_v.at[j]])
        plsc.subcore_barrier()
        pltpu.sync_copy(fn_hbm.at[pl.ds(base, _RPT), pl.ds(col, _W)], src_v)
        for j in range(nj):
            pltpu.sync_copy(src_v.at[pl.ds(j * _CH, _CH)],
                            spm.at[idx_v.at[j]], add=True)
        if cc == 0:
            @pl.when(c == 0)
            def _():
                for j in range(nj):
                    pltpu.sync_copy(one_v, spmc.at[idx_v.at[j]], add=True)
        plsc.subcore_barrier()
        for j in range(nj):
            pltpu.sync_copy(spm.at[idx_v.at[j]],
                            src_v.at[pl.ds(j * _CH, _CH)])
        pltpu.sync_copy(src_v, msum_hbm.at[pl.ds(base, _RPT), pl.ds(col, _W)])
        if cc == 0:
            @pl.when(c == 0)
            def _():
                for j in range(nj):
                    pltpu.sync_copy(spmc.at[idx_v.at[j]], cbuf_v.at[pl.ds(j * _CH, _CH)])
                pltpu.sync_copy(cbuf_v, cnt_hbm.at[pl.ds(base, _RPT)])
        plsc.subcore_barrier()


def _segment_sums(fn, lab):
    k = pl.kernel(
        _segsum_body,
        out_type=(jax.ShapeDtypeStruct((B, D), jnp.float32),
                  jax.ShapeDtypeStruct((B,), jnp.float32)),
        mesh=plsc.VectorSubcoreMesh(core_axis_name="c", subcore_axis_name="s"),
        compiler_params=pltpu.CompilerParams(use_tc_tiling_on_sc=False),
        scratch_types=[
            pltpu.VMEM((_RPT // _CH, _CH), jnp.int32),
            pltpu.VMEM((_RPT, _W), jnp.float32),
            pltpu.VMEM((_CH, _W), jnp.float32),
            pltpu.VMEM((_CH,), jnp.float32),
            pltpu.VMEM((_CH,), jnp.float32),
            pltpu.VMEM((_RPT,), jnp.float32),
            pltpu.VMEM_SHARED((N_CLASSES, _W), jnp.float32),
            pltpu.VMEM_SHARED((N_CLASSES,), jnp.float32),
        ],
    )
    return k(fn, lab)


def _epi_body(fn_ref, bc_ref, ms_ref, ct_ref, u_ref, loss_ref):
    craw = bc_ref[...]
    mean = ms_ref[...] / ct_ref[...]
    u_ref[...] = (1.0 - A_EMA) * craw + A_EMA * mean
    ss = jnp.sum(craw * craw, axis=1, keepdims=True)
    cn = craw / jnp.maximum(jnp.sqrt(ss), EPS)
    dlt = fn_ref[...] - cn
    loss_ref[...] = jnp.sum(dlt * dlt)[None, None]


def _epilogue(fn, bc, msum, cnt):
    return pl.pallas_call(
        _epi_body,
        out_shape=[
            jax.ShapeDtypeStruct((B, D), jnp.float32),
            jax.ShapeDtypeStruct((1, 1), jnp.float32),
        ],
    )(fn, bc, msum, cnt.reshape(B, 1))


def _scatter_body(cen_hbm, idx_hbm, u_hbm, out_hbm, idx_v, rows_v, sem):
    del cen_hbm
    wid = lax.axis_index("s") * _NC + lax.axis_index("c")
    base = wid * _BPW
    pltpu.sync_copy(idx_hbm.at[pl.ds(base, _BPW)], idx_v)
    pltpu.sync_copy(u_hbm.at[pl.ds(base, _BPW)], rows_v)

    def _row(r, carry):
        lv = plsc.load_gather(idx_v, [jnp.full((16,), r, jnp.int32)])
        lab = jnp.max(lv)
        pltpu.async_copy(rows_v.at[pl.ds(r, 1)],
                         out_hbm.at[pl.ds(lab, 1)], sem)
        return carry

    lax.fori_loop(0, _BPW, _row, 0)
    pltpu.make_async_copy(u_hbm.at[pl.ds(base, _BPW)], rows_v, sem).wait()


def _copy_scatter(cen, idx, u):
    k = _plmpmd._mpmd_map(
        [(plsc.VectorSubcoreMesh(core_axis_name="c", subcore_axis_name="s"),
          _scatter_body)],
        jax.ShapeDtypeStruct((N_CLASSES, D), jnp.float32),
        input_output_aliases={0: 0},
        compiler_params=pltpu.CompilerParams(needs_layout_passes=False),
        scratch_types=[
            pltpu.VMEM((_BPW,), jnp.int32),
            pltpu.VMEM((_BPW, D), jnp.float32),
            pltpu.SemaphoreType.DMA,
        ],
    )
    return k(cen, idx, u)


def kernel(features, labels, centers):
    labels = labels.astype(jnp.int32)
    fn = _normalize(features)
    bc = _gather_rows(centers, labels)
    msum, cnt = _segment_sums(fn, labels)
    u, loss2 = _epilogue(fn, bc, msum, cnt)
    new_centers = _copy_scatter(centers, labels, u)
    loss = loss2[0, 0] / (2.0 * B)
    return loss, new_centers
